# baseline (speedup 1.0000x reference)
import functools

import jax
import jax.numpy as jnp
from jax import lax
from jax.experimental import pallas as pl
from jax.experimental.pallas import tpu as pltpu
from jax.experimental.pallas import tpu_sc as plsc

N = 10000
E = 320000
D = 128
L = 3

NC = 2
NS = 16
NW = NC * NS
K = 128
EPAD = 327680
NCH = EPAD // K
CPT = NCH // NS
NR = 5000
NPH = 5376
RPT = NPH // NS

_HI = jax.lax.Precision.HIGHEST
_mesh = plsc.VectorSubcoreMesh(core_axis_name="c", subcore_axis_name="s")


def _sc_scatter(row2, col_loc, meta, g, zk):
  nb = 2

  @functools.partial(
      pl.kernel,
      mesh=_mesh,
      out_type=jax.ShapeDtypeStruct((NC, NPH, D), jnp.float32),
      scratch_types=[
          pltpu.VMEM((CPT, K), jnp.int32),
          pltpu.VMEM((CPT, K), jnp.int32),
          pltpu.VMEM((128,), jnp.int32),
          [pltpu.VMEM((K, D), jnp.float32) for _ in range(nb)],
          [pltpu.SemaphoreType.DMA for _ in range(nb)],
          [pltpu.SemaphoreType.DMA for _ in range(nb)],
          pltpu.VMEM_SHARED((NPH, D), jnp.float32),
      ],
  )
  def body(row_hbm, col_hbm, meta_hbm, g_hbm, zero_hbm, out_hbm,
           ridx, cidx, metav, bufs, gsem, ssem, acc):
    cid = lax.axis_index("c")
    sid = lax.axis_index("s")

    pltpu.sync_copy(meta_hbm.at[cid, sid, 0], metav)
    mv = metav[pl.ds(0, 16)]
    start = mv[0]
    nch = mv[1]
    base = pl.multiple_of(jnp.minimum(start, NCH - CPT), 8)
    off = start - base
    pltpu.sync_copy(row_hbm.at[pl.ds(base, CPT)], ridx)
    pltpu.sync_copy(col_hbm.at[cid, pl.ds(base, CPT)], cidx)

    pltpu.sync_copy(zero_hbm, bufs[0])
    r0 = sid * RPT
    pltpu.sync_copy(bufs[0], acc.at[pl.ds(r0, K)])
    pltpu.sync_copy(bufs[0], acc.at[pl.ds(r0 + K, K)])
    pltpu.sync_copy(bufs[0].at[pl.ds(0, RPT - 2 * K)],
                    acc.at[pl.ds(r0 + 2 * K, RPT - 2 * K)])
    plsc.subcore_barrier()

    @pl.when(nch > 0)
    def _():
      for b in range(nb):
        pltpu.async_copy(g_hbm.at[ridx.at[off + b]], bufs[b], gsem[b])

    def step(t, carry):
      for b in range(nb):
        j = nb * t + b
        pltpu.make_async_copy(g_hbm.at[ridx.at[off + j]], bufs[b],
                              gsem[b]).wait()
        pltpu.sync_copy(bufs[b], acc.at[cidx.at[off + j]], add=True)

        @pl.when(t < nch // nb - 1)
        def _():
          pltpu.async_copy(g_hbm.at[ridx.at[off + j + nb]], bufs[b], gsem[b])
      return carry

    lax.fori_loop(0, nch // nb, step, 0)
    plsc.subcore_barrier()

    pltpu.sync_copy(acc.at[pl.ds(r0, RPT)], out_hbm.at[cid, pl.ds(r0, RPT)])

  return body(row2, col_loc, meta, g, zk)


_BN = 1000


def _acc_spec():
  return pl.BlockSpec((1, _BN, D), lambda i: (i // 5, i % 5, 0))


def _dinv_body(a_ref, o_ref):
  deg = a_ref[0, :, 0:1] + 1.0
  o_ref[...] = jax.lax.rsqrt(jnp.broadcast_to(deg, o_ref.shape))


def _dinv(deg_acc):
  return pl.pallas_call(
      _dinv_body,
      grid=(N // _BN,),
      in_specs=[_acc_spec()],
      out_specs=pl.BlockSpec((_BN, D), lambda i: (i, 0)),
      out_shape=jax.ShapeDtypeStruct((N, D), jnp.float32),
  )(deg_acc)


def _g_body(x_ref, w_ref, dinv_ref, o_ref):
  h = jnp.dot(x_ref[...], w_ref[...], preferred_element_type=jnp.float32,
              precision=_HI)
  o_ref[...] = dinv_ref[...] * h


def _gcall(x, w, dinv_b):
  row = pl.BlockSpec((_BN, D), lambda i: (i, 0))
  return pl.pallas_call(
      _g_body,
      grid=(N // _BN,),
      in_specs=[row, pl.BlockSpec((D, D), lambda i: (0, 0)), row],
      out_specs=row,
      out_shape=jax.ShapeDtypeStruct((N, D), jnp.float32),
  )(x, w, dinv_b)


def _ep_body(x_ref, g_ref, acc_ref, dinv_ref, cb_ref, w1_ref, b1_ref,
             w2_ref, b2_ref, o_ref):
  z = dinv_ref[...] * (acc_ref[0] + g_ref[...]) + cb_ref[...]
  z = jnp.maximum(z, 0.0)
  t = jnp.dot(z, w1_ref[...], preferred_element_type=jnp.float32,
              precision=_HI) + b1_ref[...]
  t = jnp.maximum(t, 0.0)
  r = jnp.dot(t, w2_ref[...], preferred_element_type=jnp.float32,
              precision=_HI) + b2_ref[...]
  o_ref[...] = x_ref[...] + r


def _epilogue(x, g, acc, dinv_b, cb, w1, b1, w2, b2):
  mat = pl.BlockSpec((D, D), lambda i: (0, 0))
  vec = pl.BlockSpec((1, D), lambda i: (0, 0))
  row = pl.BlockSpec((_BN, D), lambda i: (i, 0))
  return pl.pallas_call(
      _ep_body,
      grid=(N // _BN,),
      in_specs=[row, row, _acc_spec(), row, vec, mat, vec, mat, vec],
      out_specs=row,
      out_shape=jax.ShapeDtypeStruct((N, D), jnp.float32),
  )(x, g, acc, dinv_b, cb, w1, b1, w2, b2)


def _epf_body(x_ref, g_ref, acc_ref, dinv_ref, cb_ref, w1_ref, b1_ref,
              w2_ref, b2_ref, wn_ref, ox_ref, og_ref):
  z = dinv_ref[...] * (acc_ref[0] + g_ref[...]) + cb_ref[...]
  z = jnp.maximum(z, 0.0)
  t = jnp.dot(z, w1_ref[...], preferred_element_type=jnp.float32,
              precision=_HI) + b1_ref[...]
  t = jnp.maximum(t, 0.0)
  r = jnp.dot(t, w2_ref[...], preferred_element_type=jnp.float32,
              precision=_HI) + b2_ref[...]
  xo = x_ref[...] + r
  ox_ref[...] = xo
  og_ref[...] = dinv_ref[...] * jnp.dot(
      xo, wn_ref[...], preferred_element_type=jnp.float32, precision=_HI)


def _ep_fused(x, g, acc, dinv_b, cb, w1, b1, w2, b2, wn):
  mat = pl.BlockSpec((D, D), lambda i: (0, 0))
  vec = pl.BlockSpec((1, D), lambda i: (0, 0))
  row = pl.BlockSpec((_BN, D), lambda i: (i, 0))
  return pl.pallas_call(
      _epf_body,
      grid=(N // _BN,),
      in_specs=[row, row, _acc_spec(), row, vec, mat, vec, mat, vec, mat],
      out_specs=[row, row],
      out_shape=[jax.ShapeDtypeStruct((N, D), jnp.float32),
                 jax.ShapeDtypeStruct((N, D), jnp.float32)],
  )(x, g, acc, dinv_b, cb, w1, b1, w2, b2, wn)


def kernel(x, edge_index, convW, convB, refW1, refB1, refW2, refB2):
  row = edge_index[0]
  col = edge_index[1]
  pad = EPAD - E
  rowp = jnp.concatenate([row, jnp.zeros((pad,), jnp.int32)])
  colp = jnp.concatenate([col, jnp.full((pad,), N, jnp.int32)])
  n0 = jnp.sum(colp < NR)
  n1 = jnp.sum((colp >= NR) & (colp < N))
  packed = jax.lax.sort((colp << 14) | rowp)
  colp = packed >> 14
  rowp = packed & 16383
  dummy = NR + (colp & 255)
  loc0 = jnp.where(colp < NR, colp, dummy)
  loc1 = jnp.where((colp >= NR) & (colp < N), colp - NR, dummy)
  row2 = rowp.reshape(NCH, K)
  col_loc = jnp.stack([loc0, loc1]).reshape(NC, NCH, K)

  G = 8
  ghi0 = (n0 + K * G - 1) // (K * G)
  glo1 = n0 // (K * G)
  ghi1 = (n0 + n1 + K * G - 1) // (K * G)
  sidx = jnp.arange(NS + 1)

  def spans(glo, ghi):
    bounds = glo + (sidx * (ghi - glo)) // NS
    return bounds[:-1] * G, (bounds[1:] - bounds[:-1]) * G

  st0, ct0 = spans(0, ghi0)
  st1, ct1 = spans(glo1, ghi1)
  meta2 = jnp.stack([jnp.stack([st0, st1]), jnp.stack([ct0, ct1])],
                    axis=-1).astype(jnp.int32)
  meta = jnp.pad(meta2, ((0, 0), (0, 0), (0, 126)))[:, :, None, :]

  zk = jnp.zeros((K, D), jnp.float32)
  ones_g = jnp.ones((N, D), jnp.float32)

  deg_acc = _sc_scatter(row2, col_loc, meta, ones_g, zk)
  dinv_b = _dinv(deg_acc)

  g = _gcall(x, convW[0], dinv_b)
  for i in range(L):
    acc = _sc_scatter(row2, col_loc, meta, g, zk)
    args = (x, g, acc, dinv_b, convB[i].reshape(1, D), refW1[i],
            refB1[i].reshape(1, D), refW2[i], refB2[i].reshape(1, D))
    if i + 1 < L:
      x, g = _ep_fused(*args, convW[i + 1])
    else:
      x = _epilogue(*args)
  return x


# speedup vs baseline: 1.0167x; 1.0167x over previous
import functools

import jax
import jax.numpy as jnp
from jax import lax
from jax.experimental import pallas as pl
from jax.experimental.pallas import tpu as pltpu
from jax.experimental.pallas import tpu_sc as plsc

N = 10000
E = 320000
D = 128
L = 3

NC = 2
NS = 16
NW = NC * NS
K = 128
EPAD = 327680
NCH = EPAD // K
CPT = NCH // NS
NR = 5000
NPH = 5376
RPT = NPH // NS

_HI = jax.lax.Precision.HIGHEST
_mesh = plsc.VectorSubcoreMesh(core_axis_name="c", subcore_axis_name="s")


def _sc_scatter(row2, col_loc, meta, g, zk):
  nb = 2

  @functools.partial(
      pl.kernel,
      mesh=_mesh,
      out_type=jax.ShapeDtypeStruct((NC, NPH, D), jnp.float32),
      scratch_types=[
          pltpu.VMEM((CPT, K), jnp.int32),
          pltpu.VMEM((CPT, K), jnp.int32),
          pltpu.VMEM((128,), jnp.int32),
          [pltpu.VMEM((K, D), jnp.float32) for _ in range(nb)],
          [pltpu.SemaphoreType.DMA for _ in range(nb)],
          [pltpu.SemaphoreType.DMA for _ in range(nb)],
          pltpu.VMEM_SHARED((NPH, D), jnp.float32),
      ],
  )
  def body(row_hbm, col_hbm, meta_hbm, g_hbm, zero_hbm, out_hbm,
           ridx, cidx, metav, bufs, gsem, ssem, acc):
    cid = lax.axis_index("c")
    sid = lax.axis_index("s")

    pltpu.sync_copy(meta_hbm.at[cid, sid, 0], metav)
    mv = metav[pl.ds(0, 16)]
    start = mv[0]
    nch = mv[1]
    base = pl.multiple_of(jnp.minimum(start, NCH - CPT), 8)
    off = start - base
    pltpu.sync_copy(row_hbm.at[pl.ds(base, CPT)], ridx)
    pltpu.sync_copy(col_hbm.at[cid, pl.ds(base, CPT)], cidx)

    pltpu.sync_copy(zero_hbm, bufs[0])
    r0 = sid * RPT
    pltpu.sync_copy(bufs[0], acc.at[pl.ds(r0, K)])
    pltpu.sync_copy(bufs[0], acc.at[pl.ds(r0 + K, K)])
    pltpu.sync_copy(bufs[0].at[pl.ds(0, RPT - 2 * K)],
                    acc.at[pl.ds(r0 + 2 * K, RPT - 2 * K)])
    plsc.subcore_barrier()

    @pl.when(nch > 0)
    def _():
      for b in range(nb):
        pltpu.async_copy(g_hbm.at[ridx.at[off + b]], bufs[b], gsem[b])

    def step(t, carry):
      for b in range(nb):
        j = nb * t + b
        pltpu.make_async_copy(g_hbm.at[ridx.at[off + j]], bufs[b],
                              gsem[b]).wait()
        pltpu.sync_copy(bufs[b], acc.at[cidx.at[off + j]], add=True)

        @pl.when(t < nch // nb - 1)
        def _():
          pltpu.async_copy(g_hbm.at[ridx.at[off + j + nb]], bufs[b], gsem[b])
      return carry

    lax.fori_loop(0, nch // nb, step, 0)
    plsc.subcore_barrier()

    pltpu.sync_copy(acc.at[pl.ds(r0, RPT)], out_hbm.at[cid, pl.ds(r0, RPT)])

  return body(row2, col_loc, meta, g, zk)


_BN = 1000


def _acc_spec():
  return pl.BlockSpec((1, _BN, D), lambda i: (i // 5, i % 5, 0))


def _dinv_body(a_ref, o_ref):
  deg = a_ref[0, :, 0:1] + 1.0
  o_ref[...] = jax.lax.rsqrt(jnp.broadcast_to(deg, o_ref.shape))


def _dinv(deg_acc):
  return pl.pallas_call(
      _dinv_body,
      grid=(N // _BN,),
      in_specs=[_acc_spec()],
      out_specs=pl.BlockSpec((_BN, D), lambda i: (i, 0)),
      out_shape=jax.ShapeDtypeStruct((N, D), jnp.float32),
  )(deg_acc)


def _g_body(x_ref, w_ref, dinv_ref, o_ref):
  h = jnp.dot(x_ref[...], w_ref[...], preferred_element_type=jnp.float32,
              precision=_HI)
  o_ref[...] = dinv_ref[...] * h


def _gcall(x, w, dinv_b):
  row = pl.BlockSpec((_BN, D), lambda i: (i, 0))
  return pl.pallas_call(
      _g_body,
      grid=(N // _BN,),
      in_specs=[row, pl.BlockSpec((D, D), lambda i: (0, 0)), row],
      out_specs=row,
      out_shape=jax.ShapeDtypeStruct((N, D), jnp.float32),
  )(x, w, dinv_b)


def _ep_body(x_ref, g_ref, acc_ref, dinv_ref, cb_ref, w1_ref, b1_ref,
             w2_ref, b2_ref, o_ref):
  z = dinv_ref[...] * (acc_ref[0] + g_ref[...]) + cb_ref[...]
  z = jnp.maximum(z, 0.0)
  t = jnp.dot(z, w1_ref[...], preferred_element_type=jnp.float32,
              precision=_HI) + b1_ref[...]
  t = jnp.maximum(t, 0.0)
  r = jnp.dot(t, w2_ref[...], preferred_element_type=jnp.float32,
              precision=_HI) + b2_ref[...]
  o_ref[...] = x_ref[...] + r


def _epilogue(x, g, acc, dinv_b, cb, w1, b1, w2, b2):
  mat = pl.BlockSpec((D, D), lambda i: (0, 0))
  vec = pl.BlockSpec((1, D), lambda i: (0, 0))
  row = pl.BlockSpec((_BN, D), lambda i: (i, 0))
  return pl.pallas_call(
      _ep_body,
      grid=(N // _BN,),
      in_specs=[row, row, _acc_spec(), row, vec, mat, vec, mat, vec],
      out_specs=row,
      out_shape=jax.ShapeDtypeStruct((N, D), jnp.float32),
  )(x, g, acc, dinv_b, cb, w1, b1, w2, b2)


def _epf_body(x_ref, g_ref, acc_ref, dinv_ref, cb_ref, w1_ref, b1_ref,
              w2_ref, b2_ref, wn_ref, ox_ref, og_ref):
  z = dinv_ref[...] * (acc_ref[0] + g_ref[...]) + cb_ref[...]
  z = jnp.maximum(z, 0.0)
  t = jnp.dot(z, w1_ref[...], preferred_element_type=jnp.float32,
              precision=_HI) + b1_ref[...]
  t = jnp.maximum(t, 0.0)
  r = jnp.dot(t, w2_ref[...], preferred_element_type=jnp.float32,
              precision=_HI) + b2_ref[...]
  xo = x_ref[...] + r
  ox_ref[...] = xo
  og_ref[...] = dinv_ref[...] * jnp.dot(
      xo, wn_ref[...], preferred_element_type=jnp.float32, precision=_HI)


def _ep_fused(x, g, acc, dinv_b, cb, w1, b1, w2, b2, wn):
  mat = pl.BlockSpec((D, D), lambda i: (0, 0))
  vec = pl.BlockSpec((1, D), lambda i: (0, 0))
  row = pl.BlockSpec((_BN, D), lambda i: (i, 0))
  return pl.pallas_call(
      _epf_body,
      grid=(N // _BN,),
      in_specs=[row, row, _acc_spec(), row, vec, mat, vec, mat, vec, mat],
      out_specs=[row, row],
      out_shape=[jax.ShapeDtypeStruct((N, D), jnp.float32),
                 jax.ShapeDtypeStruct((N, D), jnp.float32)],
  )(x, g, acc, dinv_b, cb, w1, b1, w2, b2, wn)


def kernel(x, edge_index, convW, convB, refW1, refB1, refW2, refB2):
  row = edge_index[0]
  col = edge_index[1]
  pad = EPAD - E
  rowp = jnp.concatenate([row, jnp.zeros((pad,), jnp.int32)])
  colp = jnp.concatenate([col, jnp.full((pad,), N, jnp.int32)])
  keys = (colp >= NR).astype(jnp.int32) + (colp >= N).astype(jnp.int32)
  n0 = jnp.sum(keys == 0)
  n1 = jnp.sum(keys == 1)
  _, rowp, colp = jax.lax.sort([keys, rowp, colp], num_keys=1,
                               is_stable=False)
  dummy = NR + (colp & 255)
  loc0 = jnp.where(colp < NR, colp, dummy)
  loc1 = jnp.where((colp >= NR) & (colp < N), colp - NR, dummy)
  row2 = rowp.reshape(NCH, K)
  col_loc = jnp.stack([loc0, loc1]).reshape(NC, NCH, K)

  G = 8
  ghi0 = (n0 + K * G - 1) // (K * G)
  glo1 = n0 // (K * G)
  ghi1 = (n0 + n1 + K * G - 1) // (K * G)
  sidx = jnp.arange(NS + 1)

  def spans(glo, ghi):
    bounds = glo + (sidx * (ghi - glo)) // NS
    return bounds[:-1] * G, (bounds[1:] - bounds[:-1]) * G

  st0, ct0 = spans(0, ghi0)
  st1, ct1 = spans(glo1, ghi1)
  meta2 = jnp.stack([jnp.stack([st0, st1]), jnp.stack([ct0, ct1])],
                    axis=-1).astype(jnp.int32)
  meta = jnp.pad(meta2, ((0, 0), (0, 0), (0, 126)))[:, :, None, :]

  zk = jnp.zeros((K, D), jnp.float32)
  ones_g = jnp.ones((N, D), jnp.float32)

  deg_acc = _sc_scatter(row2, col_loc, meta, ones_g, zk)
  dinv_b = _dinv(deg_acc)

  g = _gcall(x, convW[0], dinv_b)
  for i in range(L):
    acc = _sc_scatter(row2, col_loc, meta, g, zk)
    args = (x, g, acc, dinv_b, convB[i].reshape(1, D), refW1[i],
            refB1[i].reshape(1, D), refW2[i], refB2[i].reshape(1, D))
    if i + 1 < L:
      x, g = _ep_fused(*args, convW[i + 1])
    else:
      x = _epilogue(*args)
  return x


# speedup vs baseline: 1.1002x; 1.0820x over previous
import functools

import jax
import jax.numpy as jnp
from jax import lax
from jax.experimental import pallas as pl
from jax.experimental.pallas import tpu as pltpu
from jax.experimental.pallas import tpu_sc as plsc

N = 10000
E = 320000
D = 128
L = 3

NC = 2
NS = 16
NW = NC * NS
K = 128
EPAD = 327680
NCH = EPAD // K
CPT = NCH // NS
NR = 5000
NPH = 5376
RPT = NPH // NS

_HI = jax.lax.Precision.HIGHEST
_mesh = plsc.VectorSubcoreMesh(core_axis_name="c", subcore_axis_name="s")


def _sc_scatter(row2, col_loc, meta, g, zk):
  nb = 2

  @functools.partial(
      pl.kernel,
      mesh=_mesh,
      out_type=jax.ShapeDtypeStruct((NC, NPH, D), jnp.float32),
      scratch_types=[
          pltpu.VMEM((CPT, K), jnp.int32),
          pltpu.VMEM((CPT, K), jnp.int32),
          pltpu.VMEM((128,), jnp.int32),
          [pltpu.VMEM((K, D), jnp.float32) for _ in range(nb)],
          [pltpu.SemaphoreType.DMA for _ in range(nb)],
          [pltpu.SemaphoreType.DMA for _ in range(nb)],
          pltpu.VMEM_SHARED((NPH, D), jnp.float32),
      ],
  )
  def body(row_hbm, col_hbm, meta_hbm, g_hbm, zero_hbm, out_hbm,
           ridx, cidx, metav, bufs, gsem, ssem, acc):
    cid = lax.axis_index("c")
    sid = lax.axis_index("s")

    pltpu.sync_copy(meta_hbm.at[cid, sid, 0], metav)
    mv = metav[pl.ds(0, 16)]
    start = mv[0]
    nch = mv[1]
    base = pl.multiple_of(jnp.minimum(start, NCH - CPT), 8)
    off = start - base
    pltpu.sync_copy(row_hbm.at[pl.ds(base, CPT)], ridx)
    pltpu.sync_copy(col_hbm.at[cid, pl.ds(base, CPT)], cidx)

    pltpu.sync_copy(zero_hbm, bufs[0])
    r0 = sid * RPT
    pltpu.sync_copy(bufs[0], acc.at[pl.ds(r0, K)])
    pltpu.sync_copy(bufs[0], acc.at[pl.ds(r0 + K, K)])
    pltpu.sync_copy(bufs[0].at[pl.ds(0, RPT - 2 * K)],
                    acc.at[pl.ds(r0 + 2 * K, RPT - 2 * K)])
    plsc.subcore_barrier()

    @pl.when(nch > 0)
    def _():
      for b in range(nb):
        pltpu.async_copy(g_hbm.at[ridx.at[off + b]], bufs[b], gsem[b])

    def step(t, carry):
      for b in range(nb):
        j = nb * t + b
        pltpu.make_async_copy(g_hbm.at[ridx.at[off + j]], bufs[b],
                              gsem[b]).wait()
        pltpu.sync_copy(bufs[b], acc.at[cidx.at[off + j]], add=True)

        @pl.when(t < nch // nb - 1)
        def _():
          pltpu.async_copy(g_hbm.at[ridx.at[off + j + nb]], bufs[b], gsem[b])
      return carry

    lax.fori_loop(0, nch // nb, step, 0)
    plsc.subcore_barrier()

    pltpu.sync_copy(acc.at[pl.ds(r0, RPT)], out_hbm.at[cid, pl.ds(r0, RPT)])

  return body(row2, col_loc, meta, g, zk)


_BN = 1000


def _acc_spec():
  return pl.BlockSpec((1, _BN, D), lambda i: (i // 5, i % 5, 0))


def _dinv_body(a_ref, o_ref):
  deg = a_ref[0, :, 0:1] + 1.0
  o_ref[...] = jax.lax.rsqrt(jnp.broadcast_to(deg, o_ref.shape))


def _dinv(deg_acc):
  return pl.pallas_call(
      _dinv_body,
      grid=(N // _BN,),
      in_specs=[_acc_spec()],
      out_specs=pl.BlockSpec((_BN, D), lambda i: (i, 0)),
      out_shape=jax.ShapeDtypeStruct((N, D), jnp.float32),
  )(deg_acc)


def _g_body(x_ref, w_ref, dinv_ref, o_ref):
  h = jnp.dot(x_ref[...], w_ref[...], preferred_element_type=jnp.float32,
              precision=_HI)
  o_ref[...] = dinv_ref[...] * h


def _gcall(x, w, dinv_b):
  row = pl.BlockSpec((_BN, D), lambda i: (i, 0))
  return pl.pallas_call(
      _g_body,
      grid=(N // _BN,),
      in_specs=[row, pl.BlockSpec((D, D), lambda i: (0, 0)), row],
      out_specs=row,
      out_shape=jax.ShapeDtypeStruct((N, D), jnp.float32),
  )(x, w, dinv_b)


def _ep_body(x_ref, g_ref, acc_ref, dinv_ref, cb_ref, w1_ref, b1_ref,
             w2_ref, b2_ref, o_ref):
  z = dinv_ref[...] * (acc_ref[0] + g_ref[...]) + cb_ref[...]
  z = jnp.maximum(z, 0.0)
  t = jnp.dot(z, w1_ref[...], preferred_element_type=jnp.float32,
              precision=_HI) + b1_ref[...]
  t = jnp.maximum(t, 0.0)
  r = jnp.dot(t, w2_ref[...], preferred_element_type=jnp.float32,
              precision=_HI) + b2_ref[...]
  o_ref[...] = x_ref[...] + r


def _epilogue(x, g, acc, dinv_b, cb, w1, b1, w2, b2):
  mat = pl.BlockSpec((D, D), lambda i: (0, 0))
  vec = pl.BlockSpec((1, D), lambda i: (0, 0))
  row = pl.BlockSpec((_BN, D), lambda i: (i, 0))
  return pl.pallas_call(
      _ep_body,
      grid=(N // _BN,),
      in_specs=[row, row, _acc_spec(), row, vec, mat, vec, mat, vec],
      out_specs=row,
      out_shape=jax.ShapeDtypeStruct((N, D), jnp.float32),
  )(x, g, acc, dinv_b, cb, w1, b1, w2, b2)


def _epf_body(x_ref, g_ref, acc_ref, dinv_ref, cb_ref, w1_ref, b1_ref,
              w2_ref, b2_ref, wn_ref, ox_ref, og_ref):
  z = dinv_ref[...] * (acc_ref[0] + g_ref[...]) + cb_ref[...]
  z = jnp.maximum(z, 0.0)
  t = jnp.dot(z, w1_ref[...], preferred_element_type=jnp.float32,
              precision=_HI) + b1_ref[...]
  t = jnp.maximum(t, 0.0)
  r = jnp.dot(t, w2_ref[...], preferred_element_type=jnp.float32,
              precision=_HI) + b2_ref[...]
  xo = x_ref[...] + r
  ox_ref[...] = xo
  og_ref[...] = dinv_ref[...] * jnp.dot(
      xo, wn_ref[...], preferred_element_type=jnp.float32, precision=_HI)


def _ep_fused(x, g, acc, dinv_b, cb, w1, b1, w2, b2, wn):
  mat = pl.BlockSpec((D, D), lambda i: (0, 0))
  vec = pl.BlockSpec((1, D), lambda i: (0, 0))
  row = pl.BlockSpec((_BN, D), lambda i: (i, 0))
  return pl.pallas_call(
      _epf_body,
      grid=(N // _BN,),
      in_specs=[row, row, _acc_spec(), row, vec, mat, vec, mat, vec, mat],
      out_specs=[row, row],
      out_shape=[jax.ShapeDtypeStruct((N, D), jnp.float32),
                 jax.ShapeDtypeStruct((N, D), jnp.float32)],
  )(x, g, acc, dinv_b, cb, w1, b1, w2, b2, wn)


def kernel(x, edge_index, convW, convB, refW1, refB1, refW2, refB2):
  row = edge_index[0]
  col = edge_index[1]
  pad = EPAD - E
  rowp = jnp.concatenate([row, jnp.zeros((pad,), jnp.int32)])
  colp = jnp.concatenate([col, jnp.full((pad,), N, jnp.int32)])
  keys = (colp >= NR).astype(jnp.int32) + (colp >= N).astype(jnp.int32)
  n0 = jnp.sum(keys == 0)
  n1 = jnp.sum(keys == 1)
  _, packed = jax.lax.sort([keys, (colp << 14) | rowp], num_keys=1,
                           is_stable=False)
  colp = packed >> 14
  rowp = packed & 16383
  dummy = NR + (colp & 255)
  loc0 = jnp.where(colp < NR, colp, dummy)
  loc1 = jnp.where((colp >= NR) & (colp < N), colp - NR, dummy)
  row2 = rowp.reshape(NCH, K)
  col_loc = jnp.stack([loc0, loc1]).reshape(NC, NCH, K)

  G = 8
  ghi0 = (n0 + K * G - 1) // (K * G)
  glo1 = n0 // (K * G)
  ghi1 = (n0 + n1 + K * G - 1) // (K * G)
  sidx = jnp.arange(NS + 1)

  def spans(glo, ghi):
    bounds = glo + (sidx * (ghi - glo)) // NS
    return bounds[:-1] * G, (bounds[1:] - bounds[:-1]) * G

  st0, ct0 = spans(0, ghi0)
  st1, ct1 = spans(glo1, ghi1)
  meta2 = jnp.stack([jnp.stack([st0, st1]), jnp.stack([ct0, ct1])],
                    axis=-1).astype(jnp.int32)
  meta = jnp.pad(meta2, ((0, 0), (0, 0), (0, 126)))[:, :, None, :]

  zk = jnp.zeros((K, D), jnp.float32)
  ones_g = jnp.ones((N, D), jnp.float32)

  deg_acc = _sc_scatter(row2, col_loc, meta, ones_g, zk)
  dinv_b = _dinv(deg_acc)

  g = _gcall(x, convW[0], dinv_b)
  for i in range(L):
    acc = _sc_scatter(row2, col_loc, meta, g, zk)
    args = (x, g, acc, dinv_b, convB[i].reshape(1, D), refW1[i],
            refB1[i].reshape(1, D), refW2[i], refB2[i].reshape(1, D))
    if i + 1 < L:
      x, g = _ep_fused(*args, convW[i + 1])
    else:
      x = _epilogue(*args)
  return x


# speedup vs baseline: 1.1246x; 1.0222x over previous
import functools

import jax
import jax.numpy as jnp
from jax import lax
from jax.experimental import pallas as pl
from jax.experimental.pallas import tpu as pltpu
from jax.experimental.pallas import tpu_sc as plsc

N = 10000
E = 320000
D = 128
L = 3

NC = 2
NS = 16
NW = NC * NS
K = 128
EPAD = 327680
NCH = EPAD // K
CPT = NCH // NS
NR = 5000
NPH = 5376
RPT = NPH // NS

_HI = jax.lax.Precision.HIGHEST
_mesh = plsc.VectorSubcoreMesh(core_axis_name="c", subcore_axis_name="s")


def _sc_scatter(row2, col_loc, meta, g, zk):
  nb = 2

  @functools.partial(
      pl.kernel,
      mesh=_mesh,
      out_type=jax.ShapeDtypeStruct((NC, NPH, D), jnp.float32),
      scratch_types=[
          pltpu.VMEM((CPT, K), jnp.int32),
          pltpu.VMEM((CPT, K), jnp.int32),
          pltpu.VMEM((128,), jnp.int32),
          [pltpu.VMEM((K, D), jnp.float32) for _ in range(nb)],
          [pltpu.SemaphoreType.DMA for _ in range(nb)],
          [pltpu.SemaphoreType.DMA for _ in range(nb)],
          pltpu.VMEM_SHARED((NPH, D), jnp.float32),
      ],
  )
  def body(row_hbm, col_hbm, meta_hbm, g_hbm, zero_hbm, out_hbm,
           ridx, cidx, metav, bufs, gsem, ssem, acc):
    cid = lax.axis_index("c")
    sid = lax.axis_index("s")

    pltpu.sync_copy(meta_hbm.at[cid, sid, 0], metav)
    mv = metav[pl.ds(0, 16)]
    start = mv[0]
    nch = mv[1]
    base = pl.multiple_of(jnp.minimum(start, NCH - CPT), 8)
    off = start - base
    pltpu.sync_copy(row_hbm.at[pl.ds(base, CPT)], ridx)
    pltpu.sync_copy(col_hbm.at[cid, pl.ds(base, CPT)], cidx)

    pltpu.sync_copy(zero_hbm, bufs[0])
    r0 = sid * RPT
    pltpu.sync_copy(bufs[0], acc.at[pl.ds(r0, K)])
    pltpu.sync_copy(bufs[0], acc.at[pl.ds(r0 + K, K)])
    pltpu.sync_copy(bufs[0].at[pl.ds(0, RPT - 2 * K)],
                    acc.at[pl.ds(r0 + 2 * K, RPT - 2 * K)])
    plsc.subcore_barrier()

    @pl.when(nch > 0)
    def _():
      for b in range(nb):
        pltpu.async_copy(g_hbm.at[ridx.at[off + b]], bufs[b], gsem[b])

    def step(t, carry):
      for b in range(nb):
        j = nb * t + b
        pltpu.make_async_copy(g_hbm.at[ridx.at[off + j]], bufs[b],
                              gsem[b]).wait()
        pltpu.sync_copy(bufs[b], acc.at[cidx.at[off + j]], add=True)

        @pl.when(t < nch // nb - 1)
        def _():
          pltpu.async_copy(g_hbm.at[ridx.at[off + j + nb]], bufs[b], gsem[b])
      return carry

    lax.fori_loop(0, nch // nb, step, 0)
    plsc.subcore_barrier()

    pltpu.sync_copy(acc.at[pl.ds(r0, RPT)], out_hbm.at[cid, pl.ds(r0, RPT)])

  return body(row2, col_loc, meta, g, zk)


_BN = 1000


def _acc_spec():
  return pl.BlockSpec((1, _BN, D), lambda i: (i // 5, i % 5, 0))


def _dinv_of(deg_ref, shape):
  return jnp.broadcast_to(jax.lax.rsqrt(deg_ref[0, :, 0:1] + 1.0), shape)


def _h_body(x_ref, w_ref, o_ref):
  o_ref[...] = jnp.dot(x_ref[...], w_ref[...],
                       preferred_element_type=jnp.float32, precision=_HI)


def _hcall(x, w):
  row = pl.BlockSpec((_BN, D), lambda i: (i, 0))
  return pl.pallas_call(
      _h_body,
      grid=(N // _BN,),
      in_specs=[row, pl.BlockSpec((D, D), lambda i: (0, 0))],
      out_specs=row,
      out_shape=jax.ShapeDtypeStruct((N, D), jnp.float32),
  )(x, w)


def _scale_body(h_ref, deg_ref, o_ref):
  o_ref[...] = _dinv_of(deg_ref, h_ref.shape) * h_ref[...]


def _scale(h, deg_acc):
  row = pl.BlockSpec((_BN, D), lambda i: (i, 0))
  return pl.pallas_call(
      _scale_body,
      grid=(N // _BN,),
      in_specs=[row, _acc_spec()],
      out_specs=row,
      out_shape=jax.ShapeDtypeStruct((N, D), jnp.float32),
  )(h, deg_acc)


def _ep_body(x_ref, g_ref, acc_ref, deg_ref, cb_ref, w1_ref, b1_ref,
             w2_ref, b2_ref, o_ref):
  z = _dinv_of(deg_ref, g_ref.shape) * (acc_ref[0] + g_ref[...]) + cb_ref[...]
  z = jnp.maximum(z, 0.0)
  t = jnp.dot(z, w1_ref[...], preferred_element_type=jnp.float32,
              precision=_HI) + b1_ref[...]
  t = jnp.maximum(t, 0.0)
  r = jnp.dot(t, w2_ref[...], preferred_element_type=jnp.float32,
              precision=_HI) + b2_ref[...]
  o_ref[...] = x_ref[...] + r


def _epilogue(x, g, acc, deg_acc, cb, w1, b1, w2, b2):
  mat = pl.BlockSpec((D, D), lambda i: (0, 0))
  vec = pl.BlockSpec((1, D), lambda i: (0, 0))
  row = pl.BlockSpec((_BN, D), lambda i: (i, 0))
  return pl.pallas_call(
      _ep_body,
      grid=(N // _BN,),
      in_specs=[row, row, _acc_spec(), _acc_spec(), vec, mat, vec, mat, vec],
      out_specs=row,
      out_shape=jax.ShapeDtypeStruct((N, D), jnp.float32),
  )(x, g, acc, deg_acc, cb, w1, b1, w2, b2)


def _epf_body(x_ref, g_ref, acc_ref, deg_ref, cb_ref, w1_ref, b1_ref,
              w2_ref, b2_ref, wn_ref, ox_ref, og_ref):
  dinv = _dinv_of(deg_ref, g_ref.shape)
  z = dinv * (acc_ref[0] + g_ref[...]) + cb_ref[...]
  z = jnp.maximum(z, 0.0)
  t = jnp.dot(z, w1_ref[...], preferred_element_type=jnp.float32,
              precision=_HI) + b1_ref[...]
  t = jnp.maximum(t, 0.0)
  r = jnp.dot(t, w2_ref[...], preferred_element_type=jnp.float32,
              precision=_HI) + b2_ref[...]
  xo = x_ref[...] + r
  ox_ref[...] = xo
  og_ref[...] = dinv * jnp.dot(
      xo, wn_ref[...], preferred_element_type=jnp.float32, precision=_HI)


def _ep_fused(x, g, acc, deg_acc, cb, w1, b1, w2, b2, wn):
  mat = pl.BlockSpec((D, D), lambda i: (0, 0))
  vec = pl.BlockSpec((1, D), lambda i: (0, 0))
  row = pl.BlockSpec((_BN, D), lambda i: (i, 0))
  return pl.pallas_call(
      _epf_body,
      grid=(N // _BN,),
      in_specs=[row, row, _acc_spec(), _acc_spec(), vec, mat, vec, mat, vec,
                mat],
      out_specs=[row, row],
      out_shape=[jax.ShapeDtypeStruct((N, D), jnp.float32),
                 jax.ShapeDtypeStruct((N, D), jnp.float32)],
  )(x, g, acc, deg_acc, cb, w1, b1, w2, b2, wn)


def kernel(x, edge_index, convW, convB, refW1, refB1, refW2, refB2):
  row = edge_index[0]
  col = edge_index[1]
  pad = EPAD - E
  rowp = jnp.concatenate([row, jnp.zeros((pad,), jnp.int32)])
  colp = jnp.concatenate([col, jnp.full((pad,), N, jnp.int32)])
  keys = (colp >= NR).astype(jnp.int32) + (colp >= N).astype(jnp.int32)
  n0 = jnp.sum(keys == 0)
  n1 = jnp.sum(keys == 1)
  _, packed = jax.lax.sort([keys, (colp << 14) | rowp], num_keys=1,
                           is_stable=False)
  colp = packed >> 14
  rowp = packed & 16383
  dummy = NR + (colp & 255)
  loc0 = jnp.where(colp < NR, colp, dummy)
  loc1 = jnp.where((colp >= NR) & (colp < N), colp - NR, dummy)
  row2 = rowp.reshape(NCH, K)
  col_loc = jnp.stack([loc0, loc1]).reshape(NC, NCH, K)

  G = 8
  ghi0 = (n0 + K * G - 1) // (K * G)
  glo1 = n0 // (K * G)
  ghi1 = (n0 + n1 + K * G - 1) // (K * G)
  sidx = jnp.arange(NS + 1)

  def spans(glo, ghi):
    bounds = glo + (sidx * (ghi - glo)) // NS
    return bounds[:-1] * G, (bounds[1:] - bounds[:-1]) * G

  st0, ct0 = spans(0, ghi0)
  st1, ct1 = spans(glo1, ghi1)
  meta2 = jnp.stack([jnp.stack([st0, st1]), jnp.stack([ct0, ct1])],
                    axis=-1).astype(jnp.int32)
  meta = jnp.pad(meta2, ((0, 0), (0, 0), (0, 126)))[:, :, None, :]

  zk = jnp.zeros((K, D), jnp.float32)
  ones_g = jnp.ones((N, D), jnp.float32)

  h0 = _hcall(x, convW[0])
  deg_acc = _sc_scatter(row2, col_loc, meta, ones_g, zk)

  g = _scale(h0, deg_acc)
  for i in range(L):
    acc = _sc_scatter(row2, col_loc, meta, g, zk)
    args = (x, g, acc, deg_acc, convB[i].reshape(1, D), refW1[i],
            refB1[i].reshape(1, D), refW2[i], refB2[i].reshape(1, D))
    if i + 1 < L:
      x, g = _ep_fused(*args, convW[i + 1])
    else:
      x = _epilogue(*args)
  return x
